# baseline (speedup 1.0000x reference)
import functools

import jax
import jax.numpy as jnp
from jax import lax
from jax.experimental import pallas as pl
from jax.experimental.pallas import tpu as pltpu
from jax.experimental.pallas import tpu_sc as plsc

RADII = (0.0625, 0.125, 0.25, 0.5)
NC, NS = 2, 16
NW = NC * NS
L = 16
K = 16
N1, N2, N3, N4 = 40000, 10000, 2500, 625
P1, P2, P3, P4 = 40960, 10240, 2560, 1024


def _leaky(x):
    return jnp.where(x >= 0, x, 0.1 * x)


def _mesh():
    return plsc.VectorSubcoreMesh(core_axis_name="c", subcore_axis_name="s")


_SC_PARAMS = pltpu.CompilerParams(needs_layout_passes=False,
                                  use_tc_tiling_on_sc=False)


def _wid():
    return lax.axis_index("s") * NC + lax.axis_index("c")


def _bcast(ref, pos):
    return plsc.load_gather(ref, [jnp.zeros((L,), jnp.int32) + pos])


def _infl_call(psx, psy, psz, pdx, pdy, pdz, idxf, radius, Pd):
    Ps = psx.shape[0]
    D = Pd // NW
    B = 64
    while D % B:
        B //= 2
    assert B >= L
    nb = D // B
    ninv = -1.0 / (radius * radius)

    def body(psx_h, psy_h, psz_h, pdx_h, pdy_h, pdz_h, idx_h, out_h,
             psx_v, psy_v, psz_v, pdx_v, pdy_v, pdz_v, idx_v, out_v):
        io = lax.iota(jnp.int32, L)
        pltpu.sync_copy(psx_h, psx_v)
        pltpu.sync_copy(psy_h, psy_v)
        pltpu.sync_copy(psz_h, psz_v)
        pt0 = _wid() * D

        def batch(g, carry):
            base = pt0 + g * B
            pltpu.sync_copy(idx_h.at[pl.ds(base * K, B * K)], idx_v)
            pltpu.sync_copy(pdx_h.at[pl.ds(base, B)], pdx_v)
            pltpu.sync_copy(pdy_h.at[pl.ds(base, B)], pdy_v)
            pltpu.sync_copy(pdz_h.at[pl.ds(base, B)], pdz_v)
            for gp in range(B // L):
                xd = pdx_v[pl.ds(gp * L, L)]
                yd = pdy_v[pl.ds(gp * L, L)]
                zd = pdz_v[pl.ds(gp * L, L)]
                for k in range(K):
                    lanes = io * K + (gp * L * K + k)
                    idxk = plsc.load_gather(idx_v, [lanes])
                    dx = plsc.load_gather(psx_v, [idxk]) - xd
                    dy = plsc.load_gather(psy_v, [idxk]) - yd
                    dz = plsc.load_gather(psz_v, [idxk]) - zd
                    d2 = dx * dx + dy * dy + dz * dz
                    plsc.store_scatter(out_v, [lanes], jnp.exp(d2 * ninv))
            pltpu.sync_copy(out_v, out_h.at[pl.ds(base * K, B * K)])
            return carry

        lax.fori_loop(0, nb, batch, 0)

    return pl.kernel(
        body,
        out_type=jax.ShapeDtypeStruct((Pd * K,), jnp.float32),
        mesh=_mesh(),
        scratch_types=[
            pltpu.VMEM((Ps,), jnp.float32),
            pltpu.VMEM((Ps,), jnp.float32),
            pltpu.VMEM((Ps,), jnp.float32),
            pltpu.VMEM((B,), jnp.float32),
            pltpu.VMEM((B,), jnp.float32),
            pltpu.VMEM((B,), jnp.float32),
            pltpu.VMEM((B * K,), jnp.int32),
            pltpu.VMEM((B * K,), jnp.float32),
        ],
        compiler_params=_SC_PARAMS,
        name=f"sc_infl_{Pd}",
    )(psx, psy, psz, pdx, pdy, pdz, idxf)


def _wsum_call(x, idxf, inflf, Pd):
    C = x.shape[1]
    D = Pd // NW
    B = max(8, 2048 // C)
    nb = D // B
    G = B * K
    chunk = min(G, 128)
    nd = G // chunk

    def body(x_h, idx_h, infl_h, out_h, idx_v, infl_v, rows_v, out_v, sem):
        pt0 = _wid() * D

        def batch(g, carry):
            base = pt0 + g * B
            pltpu.sync_copy(idx_h.at[pl.ds(base * K, G)], idx_v)
            pltpu.sync_copy(infl_h.at[pl.ds(base * K, G)], infl_v)
            cps = [
                pltpu.async_copy(
                    x_h.at[idx_v.at[pl.ds(j * chunk, chunk)]],
                    rows_v.at[pl.ds(j * chunk, chunk)], sem)
                for j in range(nd)
            ]
            for cp in cps:
                cp.wait()

            def pt(n, c2):
                if C < 128:
                    accs = [jnp.zeros((L,), jnp.float32)
                            for _ in range(C // L)]
                    for k in range(K):
                        w = _bcast(infl_v, n * K + k)
                        for cc in range(C // L):
                            accs[cc] = accs[cc] + w * rows_v[n * K + k,
                                                             pl.ds(cc * L, L)]
                else:
                    def kstep(k, accs):
                        w = _bcast(infl_v, n * K + k)
                        return tuple(
                            accs[cc] + w * rows_v[n * K + k,
                                                  pl.ds(cc * L, L)]
                            for cc in range(C // L))
                    accs = lax.fori_loop(
                        0, K + (n & 0), kstep,
                        tuple(jnp.zeros((L,), jnp.float32)
                              for _ in range(C // L)))
                for cc in range(C // L):
                    out_v[n, pl.ds(cc * L, L)] = accs[cc]
                return c2

            lax.fori_loop(0, B, pt, 0)
            pltpu.sync_copy(out_v, out_h.at[pl.ds(base, B)])
            return carry

        lax.fori_loop(0, nb, batch, 0)

    return pl.kernel(
        body,
        out_type=jax.ShapeDtypeStruct((Pd, C), jnp.float32),
        mesh=_mesh(),
        scratch_types=[
            pltpu.VMEM((G,), jnp.int32),
            pltpu.VMEM((G,), jnp.float32),
            pltpu.VMEM((G, C), jnp.float32),
            pltpu.VMEM((B, C), jnp.float32),
            pltpu.SemaphoreType.DMA,
        ],
        compiler_params=_SC_PARAMS,
        name=f"sc_wsum_{Pd}_{C}",
    )(x, idxf, inflf)


def _smax_call(f, idxf, Pd):
    C = f.shape[1]
    D = Pd // NW
    B = max(4, 2048 // C)
    nb = D // B
    G = B * K
    chunk = min(G, 128)
    nd = G // chunk

    def body(f_h, idx_h, out_h, idx_v, rows_v, out_v, sem):
        pt0 = _wid() * D

        def batch(g, carry):
            base = pt0 + g * B
            pltpu.sync_copy(idx_h.at[pl.ds(base * K, G)], idx_v)
            cps = [
                pltpu.async_copy(
                    f_h.at[idx_v.at[pl.ds(j * chunk, chunk)]],
                    rows_v.at[pl.ds(j * chunk, chunk)], sem)
                for j in range(nd)
            ]
            for cp in cps:
                cp.wait()

            def pt(n, c2):
                if C < 128:
                    for cc in range(C // L):
                        acc = rows_v[n * K, pl.ds(cc * L, L)]
                        for k in range(1, K):
                            acc = jnp.maximum(
                                acc, rows_v[n * K + k, pl.ds(cc * L, L)])
                        out_v[n, pl.ds(cc * L, L)] = acc
                else:
                    def kstep(k, accs):
                        return tuple(
                            jnp.maximum(accs[cc],
                                        rows_v[n * K + k, pl.ds(cc * L, L)])
                            for cc in range(C // L))
                    accs = lax.fori_loop(
                        1, K + (n & 0), kstep,
                        tuple(rows_v[n * K, pl.ds(cc * L, L)]
                              for cc in range(C // L)))
                    for cc in range(C // L):
                        out_v[n, pl.ds(cc * L, L)] = accs[cc]
                return c2

            lax.fori_loop(0, B, pt, 0)
            pltpu.sync_copy(out_v, out_h.at[pl.ds(base, B)])
            return carry

        lax.fori_loop(0, nb, batch, 0)

    return pl.kernel(
        body,
        out_type=jax.ShapeDtypeStruct((Pd, C), jnp.float32),
        mesh=_mesh(),
        scratch_types=[
            pltpu.VMEM((G,), jnp.int32),
            pltpu.VMEM((G, C), jnp.float32),
            pltpu.VMEM((B, C), jnp.float32),
            pltpu.SemaphoreType.DMA,
        ],
        compiler_params=_SC_PARAMS,
        name=f"sc_smax_{Pd}_{C}",
    )(f, idxf)


def _conv0_call(featf, idxf, inflf, w0):
    C = w0.shape[0]
    D = P1 // NW
    B = 64
    nb = D // B

    def body(feat_h, idx_h, infl_h, w0_h, out_h,
             feat_v, idx_v, infl_v, w0_v, agg_v, out_v):
        io = lax.iota(jnp.int32, L)
        pltpu.sync_copy(feat_h, feat_v)
        pltpu.sync_copy(w0_h, w0_v)
        pt0 = _wid() * D

        def batch(g, carry):
            base = pt0 + g * B
            pltpu.sync_copy(idx_h.at[pl.ds(base * K, B * K)], idx_v)
            pltpu.sync_copy(infl_h.at[pl.ds(base * K, B * K)], infl_v)
            for gp in range(B // L):
                acc = jnp.zeros((L,), jnp.float32)
                for k in range(K):
                    lanes = io * K + (gp * L * K + k)
                    idxk = plsc.load_gather(idx_v, [lanes])
                    fv = plsc.load_gather(feat_v, [idxk])
                    wk = plsc.load_gather(infl_v, [lanes])
                    acc = acc + wk * fv
                agg_v[pl.ds(gp * L, L)] = acc
            for c in range(C):
                wc = w0_v[pl.ds(c * L, L)]
                for gp in range(B // L):
                    av = agg_v[pl.ds(gp * L, L)]
                    out_v[pl.ds(c * B + gp * L, L)] = _leaky(av * wc)
            pltpu.sync_copy(out_v, out_h.at[pl.ds(base * C, B * C)])
            return carry

        lax.fori_loop(0, nb, batch, 0)

    out = pl.kernel(
        body,
        out_type=jax.ShapeDtypeStruct((P1 * C,), jnp.float32),
        mesh=_mesh(),
        scratch_types=[
            pltpu.VMEM((P1,), jnp.float32),
            pltpu.VMEM((B * K,), jnp.int32),
            pltpu.VMEM((B * K,), jnp.float32),
            pltpu.VMEM((C * L,), jnp.float32),
            pltpu.VMEM((B,), jnp.float32),
            pltpu.VMEM((B * C,), jnp.float32),
        ],
        compiler_params=_SC_PARAMS,
        name="sc_conv0",
    )(featf, idxf, inflf, jnp.repeat(w0, L))
    return out.reshape(P1 // 64, C, 64).transpose(0, 2, 1).reshape(P1, C)


_BM = 512


def _mm_kernel(x_ref, w_ref, o_ref):
    o_ref[...] = _leaky(jnp.dot(x_ref[...], w_ref[...],
                                preferred_element_type=jnp.float32))


def _bm(n):
    return _BM if n % _BM == 0 else 256


def _mm_act(x, W):
    n, cin = x.shape
    cout = W.shape[1]
    bm = _bm(n)
    return pl.pallas_call(
        _mm_kernel,
        grid=(n // bm,),
        in_specs=[
            pl.BlockSpec((bm, cin), lambda i: (i, 0)),
            pl.BlockSpec((cin, cout), lambda i: (0, 0)),
        ],
        out_specs=pl.BlockSpec((bm, cout), lambda i: (i, 0)),
        out_shape=jax.ShapeDtypeStruct((n, cout), jnp.float32),
    )(x, W)


def _post_sc_kernel(a_ref, s_ref, wk_ref, w2_ref, wsc_ref, o_ref):
    t = _leaky(jnp.dot(a_ref[...], wk_ref[...],
                       preferred_element_type=jnp.float32))
    u = jnp.dot(t, w2_ref[...], preferred_element_type=jnp.float32)
    v = jnp.dot(s_ref[...], wsc_ref[...], preferred_element_type=jnp.float32)
    o_ref[...] = _leaky(u + v)


def _post_id_kernel(a_ref, s_ref, wk_ref, w2_ref, o_ref):
    t = _leaky(jnp.dot(a_ref[...], wk_ref[...],
                       preferred_element_type=jnp.float32))
    u = jnp.dot(t, w2_ref[...], preferred_element_type=jnp.float32)
    o_ref[...] = _leaky(u + s_ref[...])


def _post(agg, sc, Wk, W2, Wsc=None):
    n, cmid = agg.shape
    cout = W2.shape[1]
    csc = sc.shape[1]
    bm = _bm(n)
    if Wsc is not None:
        return pl.pallas_call(
            _post_sc_kernel,
            grid=(n // bm,),
            in_specs=[
                pl.BlockSpec((bm, cmid), lambda i: (i, 0)),
                pl.BlockSpec((bm, csc), lambda i: (i, 0)),
                pl.BlockSpec((cmid, cmid), lambda i: (0, 0)),
                pl.BlockSpec((cmid, cout), lambda i: (0, 0)),
                pl.BlockSpec((csc, cout), lambda i: (0, 0)),
            ],
            out_specs=pl.BlockSpec((bm, cout), lambda i: (i, 0)),
            out_shape=jax.ShapeDtypeStruct((n, cout), jnp.float32),
        )(agg, sc, Wk, W2, Wsc)
    return pl.pallas_call(
        _post_id_kernel,
        grid=(n // bm,),
        in_specs=[
            pl.BlockSpec((bm, cmid), lambda i: (i, 0)),
            pl.BlockSpec((bm, csc), lambda i: (i, 0)),
            pl.BlockSpec((cmid, cmid), lambda i: (0, 0)),
            pl.BlockSpec((cmid, cout), lambda i: (0, 0)),
        ],
        out_specs=pl.BlockSpec((bm, cout), lambda i: (i, 0)),
        out_shape=jax.ShapeDtypeStruct((n, cout), jnp.float32),
    )(agg, sc, Wk, W2)


def _head_kernel(x_ref, w_ref, b_ref, o_ref):
    acc = jnp.dot(x_ref[...], w_ref[...], preferred_element_type=jnp.float32)
    o_ref[...] = _leaky(acc + b_ref[...])


def _head(x, Wf, bf):
    n, cin = x.shape
    cout = Wf.shape[1]
    bm = _bm(n)
    return pl.pallas_call(
        _head_kernel,
        grid=(n // bm,),
        in_specs=[
            pl.BlockSpec((bm, cin), lambda i: (i, 0)),
            pl.BlockSpec((cin, cout), lambda i: (0, 0)),
            pl.BlockSpec((1, cout), lambda i: (0, 0)),
        ],
        out_specs=pl.BlockSpec((bm, cout), lambda i: (i, 0)),
        out_shape=jax.ShapeDtypeStruct((n, cout), jnp.float32),
    )(x, Wf, bf[None, :])


def _pad_idx(idx, Pd, nsrc):
    n = idx.shape[0]
    pad = (jnp.arange((Pd - n) * K, dtype=jnp.int32) % nsrc).reshape(-1, K)
    return jnp.concatenate([idx.astype(jnp.int32), pad]).reshape(-1)


def _pad_pts(pts, Pd):
    n = pts.shape[0]
    p = jnp.pad(pts, ((0, Pd - n), (0, 0)))
    return p[:, 0], p[:, 1], p[:, 2]


def _resblock(p, f, infl, idxf, Pd, strided):
    x = _mm_act(f, p['W1'])
    agg = _wsum_call(x, idxf, infl, Pd)
    sc = _smax_call(f, idxf, Pd) if strided else f
    return _post(agg, sc, p['Wk'], p['W2'], p.get('Wsc'))


def kernel(features, pts1, pts2, pts3, pts4, neigh1, pool1, neigh2, pool2,
           neigh3, pool3, neigh4, params):
    p1 = _pad_pts(pts1, P1)
    p2 = _pad_pts(pts2, P2)
    p3 = _pad_pts(pts3, P3)
    p4 = _pad_pts(pts4, P4)
    ne1 = _pad_idx(neigh1, P1, N1)
    po1 = _pad_idx(pool1, P2, N1)
    ne2 = _pad_idx(neigh2, P2, N2)
    po2 = _pad_idx(pool2, P3, N2)
    ne3 = _pad_idx(neigh3, P3, N3)
    po3 = _pad_idx(pool3, P4, N3)
    ne4 = _pad_idx(neigh4, P4, N4)
    featf = jnp.pad(features[:, 0], (0, P1 - N1))

    infl1 = _infl_call(*p1, *p1, ne1, RADII[0], P1)
    inflp1 = _infl_call(*p1, *p2, po1, RADII[1], P2)
    infl2 = _infl_call(*p2, *p2, ne2, RADII[1], P2)
    inflp2 = _infl_call(*p2, *p3, po2, RADII[2], P3)
    infl3 = _infl_call(*p3, *p3, ne3, RADII[2], P3)
    inflp3 = _infl_call(*p3, *p4, po3, RADII[3], P4)
    infl4 = _infl_call(*p4, *p4, ne4, RADII[3], P4)

    f = _conv0_call(featf, ne1, infl1, params['conv0']['W0'][0])
    f = _resblock(params['b1r1'], f, infl1, ne1, P1, False)
    f = _resblock(params['b1r2'], f, infl1, ne1, P1, False)
    skip1 = f[:N1]
    f = _resblock(params['b2r1'], f, inflp1, po1, P2, True)
    f = _resblock(params['b2r2'], f, infl2, ne2, P2, False)
    f = _resblock(params['b2r3'], f, infl2, ne2, P2, False)
    skip2 = f[:N2]
    f = _resblock(params['b3r1'], f, inflp2, po2, P3, True)
    f = _resblock(params['b3r2'], f, infl3, ne3, P3, False)
    f = _resblock(params['b3r3'], f, infl3, ne3, P3, False)
    skip3 = f[:N3]
    f = _resblock(params['b4r1'], f, inflp3, po3, P4, True)
    f = _resblock(params['b4r2'], f, infl4, ne4, P4, False)
    f = _head(f, params['head']['Wf'], params['head']['bf'])
    return f[:N4], skip1, skip2, skip3


# speedup vs baseline: 1.0555x; 1.0555x over previous
import functools

import jax
import jax.numpy as jnp
from jax import lax
from jax.experimental import pallas as pl
from jax.experimental.pallas import tpu as pltpu
from jax.experimental.pallas import tpu_sc as plsc

RADII = (0.0625, 0.125, 0.25, 0.5)
NC, NS = 2, 16
NW = NC * NS
L = 16
K = 16
N1, N2, N3, N4 = 40000, 10000, 2500, 625
P1, P2, P3, P4 = 40960, 10240, 2560, 1024


def _leaky(x):
    return jnp.where(x >= 0, x, 0.1 * x)


def _mesh():
    return plsc.VectorSubcoreMesh(core_axis_name="c", subcore_axis_name="s")


_SC_PARAMS = pltpu.CompilerParams(needs_layout_passes=False,
                                  use_tc_tiling_on_sc=False)


def _wid():
    return lax.axis_index("s") * NC + lax.axis_index("c")


def _bcast(ref, pos):
    return plsc.load_gather(ref, [jnp.zeros((L,), jnp.int32) + pos])


def _infl_call(psx, psy, psz, pdx, pdy, pdz, idxf, radius, Pd):
    Ps = psx.shape[0]
    D = Pd // NW
    B = 64
    while D % B:
        B //= 2
    assert B >= L
    nb = D // B
    ninv = -1.0 / (radius * radius)

    def body(psx_h, psy_h, psz_h, pdx_h, pdy_h, pdz_h, idx_h, out_h,
             psx_v, psy_v, psz_v, pdx_v, pdy_v, pdz_v, idx_v, out_v):
        io = lax.iota(jnp.int32, L)
        pltpu.sync_copy(psx_h, psx_v)
        pltpu.sync_copy(psy_h, psy_v)
        pltpu.sync_copy(psz_h, psz_v)
        pt0 = _wid() * D

        def batch(g, carry):
            base = pt0 + g * B
            pltpu.sync_copy(idx_h.at[pl.ds(base * K, B * K)], idx_v)
            pltpu.sync_copy(pdx_h.at[pl.ds(base, B)], pdx_v)
            pltpu.sync_copy(pdy_h.at[pl.ds(base, B)], pdy_v)
            pltpu.sync_copy(pdz_h.at[pl.ds(base, B)], pdz_v)
            for gp in range(B // L):
                xd = pdx_v[pl.ds(gp * L, L)]
                yd = pdy_v[pl.ds(gp * L, L)]
                zd = pdz_v[pl.ds(gp * L, L)]
                for k in range(K):
                    lanes = io * K + (gp * L * K + k)
                    idxk = plsc.load_gather(idx_v, [lanes])
                    dx = plsc.load_gather(psx_v, [idxk]) - xd
                    dy = plsc.load_gather(psy_v, [idxk]) - yd
                    dz = plsc.load_gather(psz_v, [idxk]) - zd
                    d2 = dx * dx + dy * dy + dz * dz
                    plsc.store_scatter(out_v, [lanes], jnp.exp(d2 * ninv))
            pltpu.sync_copy(out_v, out_h.at[pl.ds(base * K, B * K)])
            return carry

        lax.fori_loop(0, nb, batch, 0)

    return pl.kernel(
        body,
        out_type=jax.ShapeDtypeStruct((Pd * K,), jnp.float32),
        mesh=_mesh(),
        scratch_types=[
            pltpu.VMEM((Ps,), jnp.float32),
            pltpu.VMEM((Ps,), jnp.float32),
            pltpu.VMEM((Ps,), jnp.float32),
            pltpu.VMEM((B,), jnp.float32),
            pltpu.VMEM((B,), jnp.float32),
            pltpu.VMEM((B,), jnp.float32),
            pltpu.VMEM((B * K,), jnp.int32),
            pltpu.VMEM((B * K,), jnp.float32),
        ],
        compiler_params=_SC_PARAMS,
        name=f"sc_infl_{Pd}",
    )(psx, psy, psz, pdx, pdy, pdz, idxf)


def _wsum_call(x, idxf, inflf, Pd):
    C = x.shape[1]
    D = Pd // NW
    B = max(8, 4096 // C)
    while D % B:
        B //= 2
    nb = D // B
    G = B * K
    chunk = min(G, 128)
    nd = G // chunk

    def body(x_h, idx_h, infl_h, out_h, idx_v, infl_v, rows_v, out_v, sem):
        pt0 = _wid() * D

        def batch(g, carry):
            base = pt0 + g * B
            pltpu.sync_copy(idx_h.at[pl.ds(base * K, G)], idx_v)
            pltpu.sync_copy(infl_h.at[pl.ds(base * K, G)], infl_v)
            cps = [
                pltpu.async_copy(
                    x_h.at[idx_v.at[pl.ds(j * chunk, chunk)]],
                    rows_v.at[pl.ds(j * chunk, chunk)], sem)
                for j in range(nd)
            ]
            for cp in cps:
                cp.wait()

            def pt(n, c2):
                if C < 128:
                    accs = [jnp.zeros((L,), jnp.float32)
                            for _ in range(C // L)]
                    for k in range(K):
                        w = _bcast(infl_v, n * K + k)
                        for cc in range(C // L):
                            accs[cc] = accs[cc] + w * rows_v[n * K + k,
                                                             pl.ds(cc * L, L)]
                else:
                    def kstep(k, accs):
                        w = _bcast(infl_v, n * K + k)
                        return tuple(
                            accs[cc] + w * rows_v[n * K + k,
                                                  pl.ds(cc * L, L)]
                            for cc in range(C // L))
                    accs = lax.fori_loop(
                        0, K + (n & 0), kstep,
                        tuple(jnp.zeros((L,), jnp.float32)
                              for _ in range(C // L)))
                for cc in range(C // L):
                    out_v[n, pl.ds(cc * L, L)] = accs[cc]
                return c2

            lax.fori_loop(0, B, pt, 0)
            pltpu.sync_copy(out_v, out_h.at[pl.ds(base, B)])
            return carry

        lax.fori_loop(0, nb, batch, 0)

    return pl.kernel(
        body,
        out_type=jax.ShapeDtypeStruct((Pd, C), jnp.float32),
        mesh=_mesh(),
        scratch_types=[
            pltpu.VMEM((G,), jnp.int32),
            pltpu.VMEM((G,), jnp.float32),
            pltpu.VMEM((G, C), jnp.float32),
            pltpu.VMEM((B, C), jnp.float32),
            pltpu.SemaphoreType.DMA,
        ],
        compiler_params=_SC_PARAMS,
        name=f"sc_wsum_{Pd}_{C}",
    )(x, idxf, inflf)


def _smax_call(f, idxf, Pd):
    C = f.shape[1]
    D = Pd // NW
    B = max(4, 4096 // C)
    while D % B:
        B //= 2
    nb = D // B
    G = B * K
    chunk = min(G, 128)
    nd = G // chunk

    def body(f_h, idx_h, out_h, idx_v, rows_v, out_v, sem):
        pt0 = _wid() * D

        def batch(g, carry):
            base = pt0 + g * B
            pltpu.sync_copy(idx_h.at[pl.ds(base * K, G)], idx_v)
            cps = [
                pltpu.async_copy(
                    f_h.at[idx_v.at[pl.ds(j * chunk, chunk)]],
                    rows_v.at[pl.ds(j * chunk, chunk)], sem)
                for j in range(nd)
            ]
            for cp in cps:
                cp.wait()

            def pt(n, c2):
                if C < 128:
                    for cc in range(C // L):
                        acc = rows_v[n * K, pl.ds(cc * L, L)]
                        for k in range(1, K):
                            acc = jnp.maximum(
                                acc, rows_v[n * K + k, pl.ds(cc * L, L)])
                        out_v[n, pl.ds(cc * L, L)] = acc
                else:
                    def kstep(k, accs):
                        return tuple(
                            jnp.maximum(accs[cc],
                                        rows_v[n * K + k, pl.ds(cc * L, L)])
                            for cc in range(C // L))
                    accs = lax.fori_loop(
                        1, K + (n & 0), kstep,
                        tuple(rows_v[n * K, pl.ds(cc * L, L)]
                              for cc in range(C // L)))
                    for cc in range(C // L):
                        out_v[n, pl.ds(cc * L, L)] = accs[cc]
                return c2

            lax.fori_loop(0, B, pt, 0)
            pltpu.sync_copy(out_v, out_h.at[pl.ds(base, B)])
            return carry

        lax.fori_loop(0, nb, batch, 0)

    return pl.kernel(
        body,
        out_type=jax.ShapeDtypeStruct((Pd, C), jnp.float32),
        mesh=_mesh(),
        scratch_types=[
            pltpu.VMEM((G,), jnp.int32),
            pltpu.VMEM((G, C), jnp.float32),
            pltpu.VMEM((B, C), jnp.float32),
            pltpu.SemaphoreType.DMA,
        ],
        compiler_params=_SC_PARAMS,
        name=f"sc_smax_{Pd}_{C}",
    )(f, idxf)


def _conv0_call(featf, idxf, inflf, w0):
    C = w0.shape[0]
    D = P1 // NW
    B = 64
    nb = D // B

    def body(feat_h, idx_h, infl_h, w0_h, out_h,
             feat_v, idx_v, infl_v, w0_v, agg_v, out_v):
        io = lax.iota(jnp.int32, L)
        pltpu.sync_copy(feat_h, feat_v)
        pltpu.sync_copy(w0_h, w0_v)
        pt0 = _wid() * D

        def batch(g, carry):
            base = pt0 + g * B
            pltpu.sync_copy(idx_h.at[pl.ds(base * K, B * K)], idx_v)
            pltpu.sync_copy(infl_h.at[pl.ds(base * K, B * K)], infl_v)
            for gp in range(B // L):
                acc = jnp.zeros((L,), jnp.float32)
                for k in range(K):
                    lanes = io * K + (gp * L * K + k)
                    idxk = plsc.load_gather(idx_v, [lanes])
                    fv = plsc.load_gather(feat_v, [idxk])
                    wk = plsc.load_gather(infl_v, [lanes])
                    acc = acc + wk * fv
                agg_v[pl.ds(gp * L, L)] = acc
            for c in range(C):
                wc = w0_v[pl.ds(c * L, L)]
                for gp in range(B // L):
                    av = agg_v[pl.ds(gp * L, L)]
                    out_v[pl.ds(c * B + gp * L, L)] = _leaky(av * wc)
            pltpu.sync_copy(out_v, out_h.at[pl.ds(base * C, B * C)])
            return carry

        lax.fori_loop(0, nb, batch, 0)

    out = pl.kernel(
        body,
        out_type=jax.ShapeDtypeStruct((P1 * C,), jnp.float32),
        mesh=_mesh(),
        scratch_types=[
            pltpu.VMEM((P1,), jnp.float32),
            pltpu.VMEM((B * K,), jnp.int32),
            pltpu.VMEM((B * K,), jnp.float32),
            pltpu.VMEM((C * L,), jnp.float32),
            pltpu.VMEM((B,), jnp.float32),
            pltpu.VMEM((B * C,), jnp.float32),
        ],
        compiler_params=_SC_PARAMS,
        name="sc_conv0",
    )(featf, idxf, inflf, jnp.repeat(w0, L))
    return out.reshape(P1 // 64, C, 64).transpose(0, 2, 1).reshape(P1, C)


_BM = 512


def _mm_kernel(x_ref, w_ref, o_ref):
    o_ref[...] = _leaky(jnp.dot(x_ref[...], w_ref[...],
                                preferred_element_type=jnp.float32))


def _bm(n):
    return _BM if n % _BM == 0 else 256


def _mm_act(x, W):
    n, cin = x.shape
    cout = W.shape[1]
    bm = _bm(n)
    return pl.pallas_call(
        _mm_kernel,
        grid=(n // bm,),
        in_specs=[
            pl.BlockSpec((bm, cin), lambda i: (i, 0)),
            pl.BlockSpec((cin, cout), lambda i: (0, 0)),
        ],
        out_specs=pl.BlockSpec((bm, cout), lambda i: (i, 0)),
        out_shape=jax.ShapeDtypeStruct((n, cout), jnp.float32),
    )(x, W)


def _post_sc_kernel(a_ref, s_ref, wk_ref, w2_ref, wsc_ref, o_ref):
    t = _leaky(jnp.dot(a_ref[...], wk_ref[...],
                       preferred_element_type=jnp.float32))
    u = jnp.dot(t, w2_ref[...], preferred_element_type=jnp.float32)
    v = jnp.dot(s_ref[...], wsc_ref[...], preferred_element_type=jnp.float32)
    o_ref[...] = _leaky(u + v)


def _post_id_kernel(a_ref, s_ref, wk_ref, w2_ref, o_ref):
    t = _leaky(jnp.dot(a_ref[...], wk_ref[...],
                       preferred_element_type=jnp.float32))
    u = jnp.dot(t, w2_ref[...], preferred_element_type=jnp.float32)
    o_ref[...] = _leaky(u + s_ref[...])


def _post(agg, sc, Wk, W2, Wsc=None):
    n, cmid = agg.shape
    cout = W2.shape[1]
    csc = sc.shape[1]
    bm = _bm(n)
    if Wsc is not None:
        return pl.pallas_call(
            _post_sc_kernel,
            grid=(n // bm,),
            in_specs=[
                pl.BlockSpec((bm, cmid), lambda i: (i, 0)),
                pl.BlockSpec((bm, csc), lambda i: (i, 0)),
                pl.BlockSpec((cmid, cmid), lambda i: (0, 0)),
                pl.BlockSpec((cmid, cout), lambda i: (0, 0)),
                pl.BlockSpec((csc, cout), lambda i: (0, 0)),
            ],
            out_specs=pl.BlockSpec((bm, cout), lambda i: (i, 0)),
            out_shape=jax.ShapeDtypeStruct((n, cout), jnp.float32),
        )(agg, sc, Wk, W2, Wsc)
    return pl.pallas_call(
        _post_id_kernel,
        grid=(n // bm,),
        in_specs=[
            pl.BlockSpec((bm, cmid), lambda i: (i, 0)),
            pl.BlockSpec((bm, csc), lambda i: (i, 0)),
            pl.BlockSpec((cmid, cmid), lambda i: (0, 0)),
            pl.BlockSpec((cmid, cout), lambda i: (0, 0)),
        ],
        out_specs=pl.BlockSpec((bm, cout), lambda i: (i, 0)),
        out_shape=jax.ShapeDtypeStruct((n, cout), jnp.float32),
    )(agg, sc, Wk, W2)


def _head_kernel(x_ref, w_ref, b_ref, o_ref):
    acc = jnp.dot(x_ref[...], w_ref[...], preferred_element_type=jnp.float32)
    o_ref[...] = _leaky(acc + b_ref[...])


def _head(x, Wf, bf):
    n, cin = x.shape
    cout = Wf.shape[1]
    bm = _bm(n)
    return pl.pallas_call(
        _head_kernel,
        grid=(n // bm,),
        in_specs=[
            pl.BlockSpec((bm, cin), lambda i: (i, 0)),
            pl.BlockSpec((cin, cout), lambda i: (0, 0)),
            pl.BlockSpec((1, cout), lambda i: (0, 0)),
        ],
        out_specs=pl.BlockSpec((bm, cout), lambda i: (i, 0)),
        out_shape=jax.ShapeDtypeStruct((n, cout), jnp.float32),
    )(x, Wf, bf[None, :])


def _pad_idx(idx, Pd, nsrc):
    n = idx.shape[0]
    pad = (jnp.arange((Pd - n) * K, dtype=jnp.int32) % nsrc).reshape(-1, K)
    return jnp.concatenate([idx.astype(jnp.int32), pad]).reshape(-1)


def _pad_pts(pts, Pd):
    n = pts.shape[0]
    p = jnp.pad(pts, ((0, Pd - n), (0, 0)))
    return p[:, 0], p[:, 1], p[:, 2]


def _resblock(p, f, infl, idxf, Pd, strided):
    x = _mm_act(f, p['W1'])
    agg = _wsum_call(x, idxf, infl, Pd)
    sc = _smax_call(f, idxf, Pd) if strided else f
    return _post(agg, sc, p['Wk'], p['W2'], p.get('Wsc'))


def kernel(features, pts1, pts2, pts3, pts4, neigh1, pool1, neigh2, pool2,
           neigh3, pool3, neigh4, params):
    p1 = _pad_pts(pts1, P1)
    p2 = _pad_pts(pts2, P2)
    p3 = _pad_pts(pts3, P3)
    p4 = _pad_pts(pts4, P4)
    ne1 = _pad_idx(neigh1, P1, N1)
    po1 = _pad_idx(pool1, P2, N1)
    ne2 = _pad_idx(neigh2, P2, N2)
    po2 = _pad_idx(pool2, P3, N2)
    ne3 = _pad_idx(neigh3, P3, N3)
    po3 = _pad_idx(pool3, P4, N3)
    ne4 = _pad_idx(neigh4, P4, N4)
    featf = jnp.pad(features[:, 0], (0, P1 - N1))

    infl1 = _infl_call(*p1, *p1, ne1, RADII[0], P1)
    inflp1 = _infl_call(*p1, *p2, po1, RADII[1], P2)
    infl2 = _infl_call(*p2, *p2, ne2, RADII[1], P2)
    inflp2 = _infl_call(*p2, *p3, po2, RADII[2], P3)
    infl3 = _infl_call(*p3, *p3, ne3, RADII[2], P3)
    inflp3 = _infl_call(*p3, *p4, po3, RADII[3], P4)
    infl4 = _infl_call(*p4, *p4, ne4, RADII[3], P4)

    f = _conv0_call(featf, ne1, infl1, params['conv0']['W0'][0])
    f = _resblock(params['b1r1'], f, infl1, ne1, P1, False)
    f = _resblock(params['b1r2'], f, infl1, ne1, P1, False)
    skip1 = f[:N1]
    f = _resblock(params['b2r1'], f, inflp1, po1, P2, True)
    f = _resblock(params['b2r2'], f, infl2, ne2, P2, False)
    f = _resblock(params['b2r3'], f, infl2, ne2, P2, False)
    skip2 = f[:N2]
    f = _resblock(params['b3r1'], f, inflp2, po2, P3, True)
    f = _resblock(params['b3r2'], f, infl3, ne3, P3, False)
    f = _resblock(params['b3r3'], f, infl3, ne3, P3, False)
    skip3 = f[:N3]
    f = _resblock(params['b4r1'], f, inflp3, po3, P4, True)
    f = _resblock(params['b4r2'], f, infl4, ne4, P4, False)
    f = _head(f, params['head']['Wf'], params['head']['bf'])
    return f[:N4], skip1, skip2, skip3


# speedup vs baseline: 1.0855x; 1.0284x over previous
import functools

import jax
import jax.numpy as jnp
from jax import lax
from jax.experimental import pallas as pl
from jax.experimental.pallas import tpu as pltpu
from jax.experimental.pallas import tpu_sc as plsc

RADII = (0.0625, 0.125, 0.25, 0.5)
NC, NS = 2, 16
NW = NC * NS
L = 16
K = 16
N1, N2, N3, N4 = 40000, 10000, 2500, 625
P1, P2, P3, P4 = 40960, 10240, 2560, 1024


def _leaky(x):
    return jnp.where(x >= 0, x, 0.1 * x)


def _mesh():
    return plsc.VectorSubcoreMesh(core_axis_name="c", subcore_axis_name="s")


_SC_PARAMS = pltpu.CompilerParams(needs_layout_passes=False,
                                  use_tc_tiling_on_sc=False)


def _wid():
    return lax.axis_index("s") * NC + lax.axis_index("c")


def _bcast(ref, pos):
    return plsc.load_gather(ref, [jnp.zeros((L,), jnp.int32) + pos])


def _infl_call(psx, psy, psz, pdx, pdy, pdz, idxf, radius, Pd):
    Ps = psx.shape[0]
    D = Pd // NW
    B = 128
    while D % B:
        B //= 2
    assert B >= L
    nb = D // B
    ninv = -1.0 / (radius * radius)

    def body(psx_h, psy_h, psz_h, pdx_h, pdy_h, pdz_h, idx_h, out_h,
             psx_v, psy_v, psz_v, pdx_v, pdy_v, pdz_v, idx_v, out_v):
        io = lax.iota(jnp.int32, L)
        pltpu.sync_copy(psx_h, psx_v)
        pltpu.sync_copy(psy_h, psy_v)
        pltpu.sync_copy(psz_h, psz_v)
        pt0 = _wid() * D

        def batch(g, carry):
            base = pt0 + g * B
            pltpu.sync_copy(idx_h.at[pl.ds(base * K, B * K)], idx_v)
            pltpu.sync_copy(pdx_h.at[pl.ds(base, B)], pdx_v)
            pltpu.sync_copy(pdy_h.at[pl.ds(base, B)], pdy_v)
            pltpu.sync_copy(pdz_h.at[pl.ds(base, B)], pdz_v)
            for gp in range(B // L):
                xd = pdx_v[pl.ds(gp * L, L)]
                yd = pdy_v[pl.ds(gp * L, L)]
                zd = pdz_v[pl.ds(gp * L, L)]
                for k in range(K):
                    lanes = io * K + (gp * L * K + k)
                    idxk = plsc.load_gather(idx_v, [lanes])
                    dx = plsc.load_gather(psx_v, [idxk]) - xd
                    dy = plsc.load_gather(psy_v, [idxk]) - yd
                    dz = plsc.load_gather(psz_v, [idxk]) - zd
                    d2 = dx * dx + dy * dy + dz * dz
                    plsc.store_scatter(out_v, [lanes], jnp.exp(d2 * ninv))
            pltpu.sync_copy(out_v, out_h.at[pl.ds(base * K, B * K)])
            return carry

        lax.fori_loop(0, nb, batch, 0)

    return pl.kernel(
        body,
        out_type=jax.ShapeDtypeStruct((Pd * K,), jnp.float32),
        mesh=_mesh(),
        scratch_types=[
            pltpu.VMEM((Ps,), jnp.float32),
            pltpu.VMEM((Ps,), jnp.float32),
            pltpu.VMEM((Ps,), jnp.float32),
            pltpu.VMEM((B,), jnp.float32),
            pltpu.VMEM((B,), jnp.float32),
            pltpu.VMEM((B,), jnp.float32),
            pltpu.VMEM((B * K,), jnp.int32),
            pltpu.VMEM((B * K,), jnp.float32),
        ],
        compiler_params=_SC_PARAMS,
        name=f"sc_infl_{Pd}",
    )(psx, psy, psz, pdx, pdy, pdz, idxf)


def _wsum_call(x, idxf, inflf, Pd):
    C = x.shape[1]
    D = Pd // NW
    B = max(8, 4096 // C)
    while D % B:
        B //= 2
    nb = D // B
    G = B * K
    chunk = min(G, 128)
    nd = G // chunk

    def body(x_h, idx_h, infl_h, out_h, idx_v, infl_v, rows_v, out_v, sem):
        pt0 = _wid() * D

        def batch(g, carry):
            base = pt0 + g * B
            pltpu.sync_copy(idx_h.at[pl.ds(base * K, G)], idx_v)
            pltpu.sync_copy(infl_h.at[pl.ds(base * K, G)], infl_v)
            cps = [
                pltpu.async_copy(
                    x_h.at[idx_v.at[pl.ds(j * chunk, chunk)]],
                    rows_v.at[pl.ds(j * chunk, chunk)], sem)
                for j in range(nd)
            ]
            for cp in cps:
                cp.wait()

            def pt(n, c2):
                if C < 128:
                    accs = [jnp.zeros((L,), jnp.float32)
                            for _ in range(C // L)]
                    for k in range(K):
                        w = _bcast(infl_v, n * K + k)
                        for cc in range(C // L):
                            accs[cc] = accs[cc] + w * rows_v[n * K + k,
                                                             pl.ds(cc * L, L)]
                else:
                    def kstep(k, accs):
                        w = _bcast(infl_v, n * K + k)
                        return tuple(
                            accs[cc] + w * rows_v[n * K + k,
                                                  pl.ds(cc * L, L)]
                            for cc in range(C // L))
                    accs = lax.fori_loop(
                        0, K + (n & 0), kstep,
                        tuple(jnp.zeros((L,), jnp.float32)
                              for _ in range(C // L)))
                for cc in range(C // L):
                    out_v[n, pl.ds(cc * L, L)] = accs[cc]
                return c2

            lax.fori_loop(0, B, pt, 0)
            pltpu.sync_copy(out_v, out_h.at[pl.ds(base, B)])
            return carry

        lax.fori_loop(0, nb, batch, 0)

    return pl.kernel(
        body,
        out_type=jax.ShapeDtypeStruct((Pd, C), jnp.float32),
        mesh=_mesh(),
        scratch_types=[
            pltpu.VMEM((G,), jnp.int32),
            pltpu.VMEM((G,), jnp.float32),
            pltpu.VMEM((G, C), jnp.float32),
            pltpu.VMEM((B, C), jnp.float32),
            pltpu.SemaphoreType.DMA,
        ],
        compiler_params=_SC_PARAMS,
        name=f"sc_wsum_{Pd}_{C}",
    )(x, idxf, inflf)


def _smax_call(f, idxf, Pd):
    C = f.shape[1]
    D = Pd // NW
    B = max(4, 4096 // C)
    while D % B:
        B //= 2
    nb = D // B
    G = B * K
    chunk = min(G, 128)
    nd = G // chunk

    def body(f_h, idx_h, out_h, idx_v, rows_v, out_v, sem):
        pt0 = _wid() * D

        def batch(g, carry):
            base = pt0 + g * B
            pltpu.sync_copy(idx_h.at[pl.ds(base * K, G)], idx_v)
            cps = [
                pltpu.async_copy(
                    f_h.at[idx_v.at[pl.ds(j * chunk, chunk)]],
                    rows_v.at[pl.ds(j * chunk, chunk)], sem)
                for j in range(nd)
            ]
            for cp in cps:
                cp.wait()

            def pt(n, c2):
                if C < 128:
                    for cc in range(C // L):
                        acc = rows_v[n * K, pl.ds(cc * L, L)]
                        for k in range(1, K):
                            acc = jnp.maximum(
                                acc, rows_v[n * K + k, pl.ds(cc * L, L)])
                        out_v[n, pl.ds(cc * L, L)] = acc
                else:
                    def kstep(k, accs):
                        return tuple(
                            jnp.maximum(accs[cc],
                                        rows_v[n * K + k, pl.ds(cc * L, L)])
                            for cc in range(C // L))
                    accs = lax.fori_loop(
                        1, K + (n & 0), kstep,
                        tuple(rows_v[n * K, pl.ds(cc * L, L)]
                              for cc in range(C // L)))
                    for cc in range(C // L):
                        out_v[n, pl.ds(cc * L, L)] = accs[cc]
                return c2

            lax.fori_loop(0, B, pt, 0)
            pltpu.sync_copy(out_v, out_h.at[pl.ds(base, B)])
            return carry

        lax.fori_loop(0, nb, batch, 0)

    return pl.kernel(
        body,
        out_type=jax.ShapeDtypeStruct((Pd, C), jnp.float32),
        mesh=_mesh(),
        scratch_types=[
            pltpu.VMEM((G,), jnp.int32),
            pltpu.VMEM((G, C), jnp.float32),
            pltpu.VMEM((B, C), jnp.float32),
            pltpu.SemaphoreType.DMA,
        ],
        compiler_params=_SC_PARAMS,
        name=f"sc_smax_{Pd}_{C}",
    )(f, idxf)


def _conv0_call(featf, idxf, inflf, w0):
    C = w0.shape[0]
    D = P1 // NW
    B = 128
    nb = D // B

    def body(feat_h, idx_h, infl_h, w0_h, out_h,
             feat_v, idx_v, infl_v, w0_v, agg_v, out_v):
        io = lax.iota(jnp.int32, L)
        pltpu.sync_copy(feat_h, feat_v)
        pltpu.sync_copy(w0_h, w0_v)
        pt0 = _wid() * D

        def batch(g, carry):
            base = pt0 + g * B
            pltpu.sync_copy(idx_h.at[pl.ds(base * K, B * K)], idx_v)
            pltpu.sync_copy(infl_h.at[pl.ds(base * K, B * K)], infl_v)
            for gp in range(B // L):
                acc = jnp.zeros((L,), jnp.float32)
                for k in range(K):
                    lanes = io * K + (gp * L * K + k)
                    idxk = plsc.load_gather(idx_v, [lanes])
                    fv = plsc.load_gather(feat_v, [idxk])
                    wk = plsc.load_gather(infl_v, [lanes])
                    acc = acc + wk * fv
                agg_v[pl.ds(gp * L, L)] = acc
            for c in range(C):
                wc = w0_v[pl.ds(c * L, L)]
                for gp in range(B // L):
                    av = agg_v[pl.ds(gp * L, L)]
                    out_v[pl.ds(c * B + gp * L, L)] = _leaky(av * wc)
            pltpu.sync_copy(out_v, out_h.at[pl.ds(base * C, B * C)])
            return carry

        lax.fori_loop(0, nb, batch, 0)

    out = pl.kernel(
        body,
        out_type=jax.ShapeDtypeStruct((P1 * C,), jnp.float32),
        mesh=_mesh(),
        scratch_types=[
            pltpu.VMEM((P1,), jnp.float32),
            pltpu.VMEM((B * K,), jnp.int32),
            pltpu.VMEM((B * K,), jnp.float32),
            pltpu.VMEM((C * L,), jnp.float32),
            pltpu.VMEM((B,), jnp.float32),
            pltpu.VMEM((B * C,), jnp.float32),
        ],
        compiler_params=_SC_PARAMS,
        name="sc_conv0",
    )(featf, idxf, inflf, jnp.repeat(w0, L))
    return out.reshape(P1 // 128, C, 128).transpose(0, 2, 1).reshape(P1, C)


_BM = 512


def _mm_kernel(x_ref, w_ref, o_ref):
    o_ref[...] = _leaky(jnp.dot(x_ref[...], w_ref[...],
                                preferred_element_type=jnp.float32))


def _bm(n):
    return _BM if n % _BM == 0 else 256


def _mm_act(x, W):
    n, cin = x.shape
    cout = W.shape[1]
    bm = _bm(n)
    return pl.pallas_call(
        _mm_kernel,
        grid=(n // bm,),
        in_specs=[
            pl.BlockSpec((bm, cin), lambda i: (i, 0)),
            pl.BlockSpec((cin, cout), lambda i: (0, 0)),
        ],
        out_specs=pl.BlockSpec((bm, cout), lambda i: (i, 0)),
        out_shape=jax.ShapeDtypeStruct((n, cout), jnp.float32),
    )(x, W)


def _post_sc_kernel(a_ref, s_ref, wk_ref, w2_ref, wsc_ref, o_ref):
    t = _leaky(jnp.dot(a_ref[...], wk_ref[...],
                       preferred_element_type=jnp.float32))
    u = jnp.dot(t, w2_ref[...], preferred_element_type=jnp.float32)
    v = jnp.dot(s_ref[...], wsc_ref[...], preferred_element_type=jnp.float32)
    o_ref[...] = _leaky(u + v)


def _post_id_kernel(a_ref, s_ref, wk_ref, w2_ref, o_ref):
    t = _leaky(jnp.dot(a_ref[...], wk_ref[...],
                       preferred_element_type=jnp.float32))
    u = jnp.dot(t, w2_ref[...], preferred_element_type=jnp.float32)
    o_ref[...] = _leaky(u + s_ref[...])


def _post(agg, sc, Wk, W2, Wsc=None):
    n, cmid = agg.shape
    cout = W2.shape[1]
    csc = sc.shape[1]
    bm = _bm(n)
    if Wsc is not None:
        return pl.pallas_call(
            _post_sc_kernel,
            grid=(n // bm,),
            in_specs=[
                pl.BlockSpec((bm, cmid), lambda i: (i, 0)),
                pl.BlockSpec((bm, csc), lambda i: (i, 0)),
                pl.BlockSpec((cmid, cmid), lambda i: (0, 0)),
                pl.BlockSpec((cmid, cout), lambda i: (0, 0)),
                pl.BlockSpec((csc, cout), lambda i: (0, 0)),
            ],
            out_specs=pl.BlockSpec((bm, cout), lambda i: (i, 0)),
            out_shape=jax.ShapeDtypeStruct((n, cout), jnp.float32),
        )(agg, sc, Wk, W2, Wsc)
    return pl.pallas_call(
        _post_id_kernel,
        grid=(n // bm,),
        in_specs=[
            pl.BlockSpec((bm, cmid), lambda i: (i, 0)),
            pl.BlockSpec((bm, csc), lambda i: (i, 0)),
            pl.BlockSpec((cmid, cmid), lambda i: (0, 0)),
            pl.BlockSpec((cmid, cout), lambda i: (0, 0)),
        ],
        out_specs=pl.BlockSpec((bm, cout), lambda i: (i, 0)),
        out_shape=jax.ShapeDtypeStruct((n, cout), jnp.float32),
    )(agg, sc, Wk, W2)


def _head_kernel(x_ref, w_ref, b_ref, o_ref):
    acc = jnp.dot(x_ref[...], w_ref[...], preferred_element_type=jnp.float32)
    o_ref[...] = _leaky(acc + b_ref[...])


def _head(x, Wf, bf):
    n, cin = x.shape
    cout = Wf.shape[1]
    bm = _bm(n)
    return pl.pallas_call(
        _head_kernel,
        grid=(n // bm,),
        in_specs=[
            pl.BlockSpec((bm, cin), lambda i: (i, 0)),
            pl.BlockSpec((cin, cout), lambda i: (0, 0)),
            pl.BlockSpec((1, cout), lambda i: (0, 0)),
        ],
        out_specs=pl.BlockSpec((bm, cout), lambda i: (i, 0)),
        out_shape=jax.ShapeDtypeStruct((n, cout), jnp.float32),
    )(x, Wf, bf[None, :])


def _pad_idx(idx, Pd, nsrc):
    n = idx.shape[0]
    pad = (jnp.arange((Pd - n) * K, dtype=jnp.int32) % nsrc).reshape(-1, K)
    return jnp.concatenate([idx.astype(jnp.int32), pad]).reshape(-1)


def _pad_pts(pts, Pd):
    n = pts.shape[0]
    p = jnp.pad(pts, ((0, Pd - n), (0, 0)))
    return p[:, 0], p[:, 1], p[:, 2]


def _resblock(p, f, infl, idxf, Pd, strided):
    x = _mm_act(f, p['W1'])
    agg = _wsum_call(x, idxf, infl, Pd)
    sc = _smax_call(f, idxf, Pd) if strided else f
    return _post(agg, sc, p['Wk'], p['W2'], p.get('Wsc'))


def kernel(features, pts1, pts2, pts3, pts4, neigh1, pool1, neigh2, pool2,
           neigh3, pool3, neigh4, params):
    p1 = _pad_pts(pts1, P1)
    p2 = _pad_pts(pts2, P2)
    p3 = _pad_pts(pts3, P3)
    p4 = _pad_pts(pts4, P4)
    ne1 = _pad_idx(neigh1, P1, N1)
    po1 = _pad_idx(pool1, P2, N1)
    ne2 = _pad_idx(neigh2, P2, N2)
    po2 = _pad_idx(pool2, P3, N2)
    ne3 = _pad_idx(neigh3, P3, N3)
    po3 = _pad_idx(pool3, P4, N3)
    ne4 = _pad_idx(neigh4, P4, N4)
    featf = jnp.pad(features[:, 0], (0, P1 - N1))

    infl1 = _infl_call(*p1, *p1, ne1, RADII[0], P1)
    inflp1 = _infl_call(*p1, *p2, po1, RADII[1], P2)
    infl2 = _infl_call(*p2, *p2, ne2, RADII[1], P2)
    inflp2 = _infl_call(*p2, *p3, po2, RADII[2], P3)
    infl3 = _infl_call(*p3, *p3, ne3, RADII[2], P3)
    inflp3 = _infl_call(*p3, *p4, po3, RADII[3], P4)
    infl4 = _infl_call(*p4, *p4, ne4, RADII[3], P4)

    f = _conv0_call(featf, ne1, infl1, params['conv0']['W0'][0])
    f = _resblock(params['b1r1'], f, infl1, ne1, P1, False)
    f = _resblock(params['b1r2'], f, infl1, ne1, P1, False)
    skip1 = f[:N1]
    f = _resblock(params['b2r1'], f, inflp1, po1, P2, True)
    f = _resblock(params['b2r2'], f, infl2, ne2, P2, False)
    f = _resblock(params['b2r3'], f, infl2, ne2, P2, False)
    skip2 = f[:N2]
    f = _resblock(params['b3r1'], f, inflp2, po2, P3, True)
    f = _resblock(params['b3r2'], f, infl3, ne3, P3, False)
    f = _resblock(params['b3r3'], f, infl3, ne3, P3, False)
    skip3 = f[:N3]
    f = _resblock(params['b4r1'], f, inflp3, po3, P4, True)
    f = _resblock(params['b4r2'], f, infl4, ne4, P4, False)
    f = _head(f, params['head']['Wf'], params['head']['bf'])
    return f[:N4], skip1, skip2, skip3
